# baseline (device time: 39412 ns/iter reference)
import jax
import jax.numpy as jnp
from jax import lax
from jax.experimental import pallas as pl
from jax.experimental.pallas import tpu as pltpu

N_DEV = 8
MASKS = (1, 3, 4)

_BFLY = (
    (0, 88, (1, 3, 4)),
    (704, 80, (3, 4, 1)),
    (1344, 88, (4, 1, 3)),
)


def kernel(x):
    m, n = x.shape

    def body(x_ref, out_ref, r1, r2, r3, send_sems, recv_sems):
        my = lax.axis_index("i")

        barrier_sem = pltpu.get_barrier_semaphore()
        for mask in MASKS:
            pl.semaphore_signal(
                barrier_sem, inc=1,
                device_id=(my ^ mask,), device_id_type=pl.DeviceIdType.MESH,
            )
        pl.semaphore_wait(barrier_sem, len(MASKS))

        def rows(b, c):
            base, r, _ = _BFLY[b]
            return pl.ds(base + c * r, r)

        descs = {}

        def start(b, src_buf, dst_buf, c, slot, target):
            d = pltpu.make_async_remote_copy(
                src_ref=src_buf.at[rows(b, c), :],
                dst_ref=dst_buf.at[rows(b, c), :],
                send_sem=send_sems.at[b, slot],
                recv_sem=recv_sems.at[b, slot],
                device_id=(target,),
                device_id_type=pl.DeviceIdType.MESH,
            )
            d.start()
            descs[(b, slot)] = d

        def masks(b):
            return _BFLY[b][2]

        B = range(len(_BFLY))

        for b in B:
            m1, m2, m3 = masks(b)
            for i, h in enumerate((m2, m2 ^ m3, 0, m3)):
                start(b, x_ref, r1, my ^ m1 ^ h, i, my ^ m1)

        for b in B:
            m1, m2, m3 = masks(b)
            descs[(b, 0)].wait_recv()
            c = my ^ m2
            out_ref[rows(b, c), :] = x_ref[rows(b, c), :] + r1[rows(b, c), :]
            start(b, out_ref, r2, c, 4, my ^ m2)
        for b in B:
            m1, m2, m3 = masks(b)
            descs[(b, 1)].wait_recv()
            c = my ^ m2 ^ m3
            out_ref[rows(b, c), :] = x_ref[rows(b, c), :] + r1[rows(b, c), :]
            start(b, out_ref, r2, c, 5, my ^ m2)

        for b in B:
            m1, m2, m3 = masks(b)
            for i, h in ((2, 0), (3, m3)):
                descs[(b, i)].wait_recv()
                c = my ^ h
                out_ref[rows(b, c), :] = x_ref[rows(b, c), :] + r1[rows(b, c), :]

        for b in B:
            m1, m2, m3 = masks(b)
            descs[(b, 4)].wait_recv()
            out_ref[rows(b, my), :] = out_ref[rows(b, my), :] + r2[rows(b, my), :]
            start(b, out_ref, r3, my, 6, my ^ m3)

        for b in B:
            m1, m2, m3 = masks(b)
            c = my ^ m3
            descs[(b, 5)].wait_recv()
            out_ref[rows(b, c), :] = out_ref[rows(b, c), :] + r2[rows(b, c), :]
            start(b, out_ref, r3, my ^ m3, 7, my ^ m3)

        for b in B:
            m1, m2, m3 = masks(b)
            c = my ^ m3
            descs[(b, 6)].wait_recv()
            descs[(b, 7)].wait_send()
            out_ref[rows(b, c), :] = out_ref[rows(b, c), :] + r3[rows(b, c), :]
            start(b, out_ref, out_ref, my ^ m3, 9, my ^ m2)
            start(b, out_ref, out_ref, my ^ m3, 11, my ^ m1)

        for b in B:
            m1, m2, m3 = masks(b)
            descs[(b, 7)].wait_recv()
            descs[(b, 6)].wait_send()
            out_ref[rows(b, my), :] = out_ref[rows(b, my), :] + r3[rows(b, my), :]
            start(b, out_ref, out_ref, my, 8, my ^ m2)
            start(b, out_ref, out_ref, my, 10, my ^ m1)

        for b in B:
            m1, m2, m3 = masks(b)
            descs[(b, 8)].wait_recv()
            start(b, out_ref, out_ref, my ^ m2, 12, my ^ m1)
        for b in B:
            m1, m2, m3 = masks(b)
            descs[(b, 9)].wait_recv()
            start(b, out_ref, out_ref, my ^ m2 ^ m3, 13, my ^ m1)

        for slot in (11, 10, 12, 13):
            for b in B:
                descs[(b, slot)].wait_recv()

        for (b, slot), d in descs.items():
            if slot not in (6, 7):
                d.wait_send()

    return pl.pallas_call(
        body,
        out_shape=jax.ShapeDtypeStruct((m, n), x.dtype),
        in_specs=[pl.BlockSpec(memory_space=pltpu.VMEM)],
        out_specs=pl.BlockSpec(memory_space=pltpu.VMEM),
        scratch_shapes=[
            pltpu.VMEM((m, n), x.dtype),
            pltpu.VMEM((m, n), x.dtype),
            pltpu.VMEM((m, n), x.dtype),
            pltpu.SemaphoreType.DMA((3, 14)),
            pltpu.SemaphoreType.DMA((3, 14)),
        ],
        compiler_params=pltpu.CompilerParams(collective_id=0),
    )(x)
